# baseline (speedup 1.0000x reference)
import functools

import jax
import jax.numpy as jnp
from jax import lax
from jax.experimental import pallas as pl
from jax.experimental.pallas import tpu as pltpu
from jax.experimental.pallas import tpu_sc as plsc

NUM_EXPERTS = 8
TOP_K = 2
D_MODEL = 1024
D_FF = 512
T = 2048

BT = 256
BM = 256
CAP = 6144
NB = CAP // BM
NW = 32
GPW = CAP // NW
GCH = 32
TPW = T // NW


def _router_body(x_ref, gate_ref, e0_ref, e1_ref, r0_ref, r1_ref,
                 w0_ref, w1_ref, cnt_ref, run_ref):
    i = pl.program_id(0)
    x = x_ref[...]

    logits = jnp.dot(x, gate_ref[...], preferred_element_type=jnp.float32)
    m = jnp.max(logits, axis=-1, keepdims=True)
    ex = jnp.exp(logits - m)
    probs = ex / jnp.sum(ex, axis=-1, keepdims=True)

    idx = lax.broadcasted_iota(jnp.int32, probs.shape, 1)
    m1 = jnp.max(probs, axis=-1, keepdims=True)
    i1 = jnp.min(jnp.where(probs == m1, idx, NUM_EXPERTS), axis=-1, keepdims=True)
    probs2 = jnp.where(idx == i1, -jnp.inf, probs)
    m2 = jnp.max(probs2, axis=-1, keepdims=True)
    i2 = jnp.min(jnp.where(probs2 == m2, idx, NUM_EXPERTS), axis=-1, keepdims=True)

    e0_ref[...] = i1
    e1_ref[...] = i2
    w0_ref[...] = m1
    w1_ref[...] = m2

    lane = lax.broadcasted_iota(jnp.int32, (BT, 128), 1)
    onehot0 = (lane == i1).astype(jnp.float32)
    onehot1 = (lane == i2).astype(jnp.float32)
    mm = onehot0 + onehot1

    row = lax.broadcasted_iota(jnp.int32, (BT, BT), 0)
    col = lax.broadcasted_iota(jnp.int32, (BT, BT), 1)
    tril = (col < row).astype(jnp.float32)
    c_in = jnp.dot(tril, mm, preferred_element_type=jnp.float32)

    @pl.when(i == 0)
    def _():
        run_ref[...] = jnp.zeros((1, 128), jnp.float32)

    ctot = c_in + run_ref[...]
    r0_ref[...] = jnp.sum(jnp.where(onehot0 > 0, ctot, 0.0),
                          axis=1, keepdims=True).astype(jnp.int32)
    r1_ref[...] = jnp.sum(jnp.where(onehot1 > 0, ctot, 0.0),
                          axis=1, keepdims=True).astype(jnp.int32)

    new_run = run_ref[...] + jnp.sum(mm, axis=0, keepdims=True)
    run_ref[...] = new_run
    cnt_ref[...] = new_run.astype(jnp.int32)


def _router(hidden_states, gate_w):
    nT = T // BT
    out_shapes = [
        jax.ShapeDtypeStruct((T, 1), jnp.int32),
        jax.ShapeDtypeStruct((T, 1), jnp.int32),
        jax.ShapeDtypeStruct((T, 1), jnp.int32),
        jax.ShapeDtypeStruct((T, 1), jnp.int32),
        jax.ShapeDtypeStruct((T, 1), jnp.float32),
        jax.ShapeDtypeStruct((T, 1), jnp.float32),
        jax.ShapeDtypeStruct((1, 128), jnp.int32),
    ]
    tok_spec = pl.BlockSpec((BT, 1), lambda i: (i, 0))
    return pl.pallas_call(
        _router_body,
        grid=(nT,),
        in_specs=[
            pl.BlockSpec((BT, D_MODEL), lambda i: (i, 0)),
            pl.BlockSpec((D_MODEL, NUM_EXPERTS), lambda i: (0, 0)),
        ],
        out_specs=[tok_spec, tok_spec, tok_spec, tok_spec, tok_spec, tok_spec,
                   pl.BlockSpec((1, 128), lambda i: (0, 0))],
        out_shape=out_shapes,
        scratch_shapes=[pltpu.VMEM((1, 128), jnp.float32)],
        compiler_params=pltpu.CompilerParams(
            dimension_semantics=("arbitrary",),
        ),
    )(hidden_states, gate_w)


def _meta_body(cnt_ref, e0_ref, e1_ref, r0_ref, r1_ref, w0_ref, w1_ref,
               pos0_ref, pos1_ref, blk_ref, tok_ref, wl_ref):
    b = pl.program_id(0)

    lane1 = lax.broadcasted_iota(jnp.int32, (1, 128), 1)
    cnt = jnp.where(lane1 < NUM_EXPERTS,
                    cnt_ref[...].astype(jnp.float32), 0.0)
    pc = jnp.ceil(cnt * (1.0 / BM)) * BM

    r_i = lax.broadcasted_iota(jnp.int32, (128, 128), 0)
    c_i = lax.broadcasted_iota(jnp.int32, (128, 128), 1)
    ut = (r_i < c_i).astype(jnp.float32)
    offs = jnp.dot(pc, ut, preferred_element_type=jnp.float32)
    ends = offs + pc

    sub = lax.broadcasted_iota(jnp.int32, (128, 128), 0)
    ends_b = jnp.broadcast_to(ends, (128, 128))
    lane2 = lax.broadcasted_iota(jnp.int32, (128, 128), 1)
    cmp = ((ends_b <= (sub * BM).astype(jnp.float32)) &
           (lane2 < NUM_EXPERTS)).astype(jnp.int32)
    blk_ref[...] = jnp.minimum(jnp.sum(cmp, axis=1, keepdims=True),
                               NUM_EXPERTS - 1)

    e0 = e0_ref[...]
    e1 = e1_ref[...]
    lane_t = lax.broadcasted_iota(jnp.int32, (T, 128), 1)
    oh0 = (lane_t == e0).astype(jnp.float32)
    oh1 = (lane_t == e1).astype(jnp.float32)
    pos0 = jnp.sum(oh0 * offs, axis=1, keepdims=True).astype(jnp.int32) + r0_ref[...]
    pos1 = jnp.sum(oh1 * offs, axis=1, keepdims=True).astype(jnp.int32) + r1_ref[...]
    pos0_ref[...] = pos0
    pos1_ref[...] = pos1

    col = lax.broadcasted_iota(jnp.int32, (T, BM), 1) + b * BM
    p0 = (jnp.broadcast_to(pos0, (T, BM)) == col).astype(jnp.float32)
    p1 = (jnp.broadcast_to(pos1, (T, BM)) == col).astype(jnp.float32)
    tv = lax.broadcasted_iota(jnp.int32, (T, 1), 0).astype(jnp.float32)
    dn = (((0,), (0,)), ((), ()))
    tok = lax.dot_general(p0 + p1, tv, dn,
                          preferred_element_type=jnp.float32)
    tok_ref[...] = tok.astype(jnp.int32)
    wl_ref[...] = (
        lax.dot_general(p0, w0_ref[...], dn, preferred_element_type=jnp.float32)
        + lax.dot_general(p1, w1_ref[...], dn, preferred_element_type=jnp.float32))


def _meta(counts, e0, e1, r0, r1, w0, w1):
    full = pl.BlockSpec((T, 1), lambda b: (0, 0))
    return pl.pallas_call(
        _meta_body,
        grid=(NB,),
        in_specs=[pl.BlockSpec((1, 128), lambda b: (0, 0)),
                  full, full, full, full, full, full],
        out_specs=[
            pl.BlockSpec((T, 1), lambda b: (0, 0)),
            pl.BlockSpec((T, 1), lambda b: (0, 0)),
            pl.BlockSpec((128, 1), lambda b: (0, 0)),
            pl.BlockSpec((BM, 1), lambda b: (b, 0)),
            pl.BlockSpec((BM, 1), lambda b: (b, 0)),
        ],
        out_shape=[
            jax.ShapeDtypeStruct((T, 1), jnp.int32),
            jax.ShapeDtypeStruct((T, 1), jnp.int32),
            jax.ShapeDtypeStruct((128, 1), jnp.int32),
            jax.ShapeDtypeStruct((CAP, 1), jnp.int32),
            jax.ShapeDtypeStruct((CAP, 1), jnp.float32),
        ],
        compiler_params=pltpu.CompilerParams(
            dimension_semantics=("arbitrary",),
        ),
    )(counts, e0, e1, r0, r1, w0, w1)


def _sc_gather_body(tok_h, x_h, xs_h, idx_v, rows_v, sem):
    wid = lax.axis_index("s") * 2 + lax.axis_index("c")
    base = wid * GPW
    for ch in range(GPW // GCH):
        b = base + ch * GCH
        pltpu.sync_copy(tok_h.at[pl.ds(b, GCH)], idx_v)
        pltpu.async_copy(x_h.at[idx_v], rows_v, sem).wait()
        pltpu.sync_copy(rows_v, xs_h.at[pl.ds(b, GCH)])


def _sc_gather(tok, x):
    mesh = plsc.VectorSubcoreMesh(core_axis_name="c", subcore_axis_name="s")
    k = functools.partial(
        pl.kernel,
        mesh=mesh,
        out_type=jax.ShapeDtypeStruct((CAP, D_MODEL), jnp.float32),
        scratch_types=[
            pltpu.VMEM((GCH,), jnp.int32),
            pltpu.VMEM((GCH, D_MODEL), jnp.float32),
            pltpu.SemaphoreType.DMA,
        ],
        compiler_params=pltpu.CompilerParams(needs_layout_passes=False),
    )(_sc_gather_body)
    return k(tok, x)


def _ffn_body(blk_ref, xs_ref, wl_ref, wgu_ref, wdn_ref, ys_ref):
    xb = xs_ref[...].astype(jnp.bfloat16)
    gu = jnp.dot(xb, wgu_ref[0], preferred_element_type=jnp.float32)
    g = gu[:, :D_FF]
    u = gu[:, D_FF:]
    act = (g * lax.logistic(g)) * u * wl_ref[...]
    ys_ref[...] = jnp.dot(act.astype(jnp.bfloat16), wdn_ref[0],
                          preferred_element_type=jnp.float32)


def _grouped_ffn(blk_e, xs2d, wl2d, wgu, wdn):
    grid_spec = pltpu.PrefetchScalarGridSpec(
        num_scalar_prefetch=1,
        grid=(NB,),
        in_specs=[
            pl.BlockSpec((BM, D_MODEL), lambda b, be: (b, 0)),
            pl.BlockSpec((BM, 1), lambda b, be: (b, 0)),
            pl.BlockSpec((1, D_MODEL, 2 * D_FF), lambda b, be: (be[b], 0, 0)),
            pl.BlockSpec((1, D_FF, D_MODEL), lambda b, be: (be[b], 0, 0)),
        ],
        out_specs=pl.BlockSpec((BM, D_MODEL), lambda b, be: (b, 0)),
    )
    return pl.pallas_call(
        _ffn_body,
        grid_spec=grid_spec,
        out_shape=jax.ShapeDtypeStruct((CAP, D_MODEL), jnp.float32),
        compiler_params=pltpu.CompilerParams(
            dimension_semantics=("arbitrary",),
        ),
    )(blk_e, xs2d, wl2d, wgu, wdn)


def _sc_combine_body(ys_h, pos0_h, pos1_h, out_h,
                     p_v, rows0_v, rows1_v, sem):
    wid = lax.axis_index("s") * 2 + lax.axis_index("c")
    for ch in range(TPW // GCH):
        b = wid * TPW + ch * GCH
        pltpu.sync_copy(pos0_h.at[pl.ds(b, GCH)], p_v)
        pltpu.async_copy(ys_h.at[p_v], rows0_v, sem).wait()
        pltpu.sync_copy(pos1_h.at[pl.ds(b, GCH)], p_v)
        pltpu.async_copy(ys_h.at[p_v], rows1_v, sem).wait()

        for i in range(GCH):
            def lane_body(j, _2):
                sl = pl.ds(j * 16, 16)
                rows0_v[i, sl] = rows0_v[i, sl] + rows1_v[i, sl]
                return 0
            lax.fori_loop(0, D_MODEL // 16, lane_body, 0)
        pltpu.sync_copy(rows0_v, out_h.at[pl.ds(b, GCH)])


def _sc_combine(ys, pos0, pos1):
    mesh = plsc.VectorSubcoreMesh(core_axis_name="c", subcore_axis_name="s")
    k = functools.partial(
        pl.kernel,
        mesh=mesh,
        out_type=jax.ShapeDtypeStruct((T, D_MODEL), jnp.float32),
        scratch_types=[
            pltpu.VMEM((GCH,), jnp.int32),
            pltpu.VMEM((GCH, D_MODEL), jnp.float32),
            pltpu.VMEM((GCH, D_MODEL), jnp.float32),
            pltpu.SemaphoreType.DMA,
        ],
        compiler_params=pltpu.CompilerParams(needs_layout_passes=False),
    )(_sc_combine_body)
    return k(ys, pos0, pos1)


def kernel(hidden_states, num_global_tokens, max_num_tokens_per_gpu,
           gate_w, w_gate_up, w_down):
    e0, e1, r0, r1, w0, w1, counts = _router(hidden_states, gate_w)

    pos0, pos1, blk, tok, wl = _meta(counts, e0, e1, r0, r1, w0, w1)

    xs = _sc_gather(tok.reshape(CAP), hidden_states)

    ys = _grouped_ffn(blk.reshape(128)[:NB], xs, wl,
                      w_gate_up.astype(jnp.bfloat16),
                      w_down.astype(jnp.bfloat16))

    return _sc_combine(ys, pos0.reshape(T), pos1.reshape(T))


# speedup vs baseline: 1.0034x; 1.0034x over previous
import functools

import jax
import jax.numpy as jnp
from jax import lax
from jax.experimental import pallas as pl
from jax.experimental.pallas import tpu as pltpu
from jax.experimental.pallas import tpu_sc as plsc

NUM_EXPERTS = 8
TOP_K = 2
D_MODEL = 1024
D_FF = 512
T = 2048

BT = 256
BM = 256
CAP = 6144
NB = CAP // BM
NW = 32
GPW = CAP // NW
GCH = 32
TPW = T // NW


def _router_body(x_ref, gate_ref, e0_ref, e1_ref, r0_ref, r1_ref,
                 w0_ref, w1_ref, cnt_ref, run_ref):
    i = pl.program_id(0)
    x = x_ref[...]

    logits = jnp.dot(x, gate_ref[...], preferred_element_type=jnp.float32)
    m = jnp.max(logits, axis=-1, keepdims=True)
    ex = jnp.exp(logits - m)
    probs = ex / jnp.sum(ex, axis=-1, keepdims=True)

    idx = lax.broadcasted_iota(jnp.int32, probs.shape, 1)
    m1 = jnp.max(probs, axis=-1, keepdims=True)
    i1 = jnp.min(jnp.where(probs == m1, idx, NUM_EXPERTS), axis=-1, keepdims=True)
    probs2 = jnp.where(idx == i1, -jnp.inf, probs)
    m2 = jnp.max(probs2, axis=-1, keepdims=True)
    i2 = jnp.min(jnp.where(probs2 == m2, idx, NUM_EXPERTS), axis=-1, keepdims=True)

    e0_ref[...] = i1
    e1_ref[...] = i2
    w0_ref[...] = m1
    w1_ref[...] = m2

    lane = lax.broadcasted_iota(jnp.int32, (BT, 128), 1)
    onehot0 = (lane == i1).astype(jnp.float32)
    onehot1 = (lane == i2).astype(jnp.float32)
    mm = onehot0 + onehot1

    row = lax.broadcasted_iota(jnp.int32, (BT, BT), 0)
    col = lax.broadcasted_iota(jnp.int32, (BT, BT), 1)
    tril = (col < row).astype(jnp.float32)
    c_in = jnp.dot(tril, mm, preferred_element_type=jnp.float32)

    @pl.when(i == 0)
    def _():
        run_ref[...] = jnp.zeros((1, 128), jnp.float32)

    ctot = c_in + run_ref[...]
    r0_ref[...] = jnp.sum(jnp.where(onehot0 > 0, ctot, 0.0),
                          axis=1, keepdims=True).astype(jnp.int32)
    r1_ref[...] = jnp.sum(jnp.where(onehot1 > 0, ctot, 0.0),
                          axis=1, keepdims=True).astype(jnp.int32)

    new_run = run_ref[...] + jnp.sum(mm, axis=0, keepdims=True)
    run_ref[...] = new_run
    cnt_ref[...] = new_run.astype(jnp.int32)


def _router(hidden_states, gate_w):
    nT = T // BT
    out_shapes = [
        jax.ShapeDtypeStruct((T, 1), jnp.int32),
        jax.ShapeDtypeStruct((T, 1), jnp.int32),
        jax.ShapeDtypeStruct((T, 1), jnp.int32),
        jax.ShapeDtypeStruct((T, 1), jnp.int32),
        jax.ShapeDtypeStruct((T, 1), jnp.float32),
        jax.ShapeDtypeStruct((T, 1), jnp.float32),
        jax.ShapeDtypeStruct((1, 128), jnp.int32),
    ]
    tok_spec = pl.BlockSpec((BT, 1), lambda i: (i, 0))
    return pl.pallas_call(
        _router_body,
        grid=(nT,),
        in_specs=[
            pl.BlockSpec((BT, D_MODEL), lambda i: (i, 0)),
            pl.BlockSpec((D_MODEL, NUM_EXPERTS), lambda i: (0, 0)),
        ],
        out_specs=[tok_spec, tok_spec, tok_spec, tok_spec, tok_spec, tok_spec,
                   pl.BlockSpec((1, 128), lambda i: (0, 0))],
        out_shape=out_shapes,
        scratch_shapes=[pltpu.VMEM((1, 128), jnp.float32)],
        compiler_params=pltpu.CompilerParams(
            dimension_semantics=("arbitrary",),
        ),
    )(hidden_states, gate_w)


def _meta_body(cnt_ref, e0_ref, e1_ref, r0_ref, r1_ref, w0_ref, w1_ref,
               pos0_ref, pos1_ref, blk_ref, tok_ref, wl_ref):
    b = pl.program_id(0)

    lane1 = lax.broadcasted_iota(jnp.int32, (1, 128), 1)
    cnt = jnp.where(lane1 < NUM_EXPERTS,
                    cnt_ref[...].astype(jnp.float32), 0.0)
    pc = jnp.ceil(cnt * (1.0 / BM)) * BM

    r_i = lax.broadcasted_iota(jnp.int32, (128, 128), 0)
    c_i = lax.broadcasted_iota(jnp.int32, (128, 128), 1)
    ut = (r_i < c_i).astype(jnp.float32)
    offs = jnp.dot(pc, ut, preferred_element_type=jnp.float32)
    ends = offs + pc

    sub = lax.broadcasted_iota(jnp.int32, (128, 128), 0)
    ends_b = jnp.broadcast_to(ends, (128, 128))
    lane2 = lax.broadcasted_iota(jnp.int32, (128, 128), 1)
    cmp = ((ends_b <= (sub * BM).astype(jnp.float32)) &
           (lane2 < NUM_EXPERTS)).astype(jnp.int32)
    blk_ref[...] = jnp.minimum(jnp.sum(cmp, axis=1, keepdims=True),
                               NUM_EXPERTS - 1)

    e0 = e0_ref[...]
    e1 = e1_ref[...]
    lane_t = lax.broadcasted_iota(jnp.int32, (T, 128), 1)
    oh0 = (lane_t == e0).astype(jnp.float32)
    oh1 = (lane_t == e1).astype(jnp.float32)
    pos0 = jnp.sum(oh0 * offs, axis=1, keepdims=True).astype(jnp.int32) + r0_ref[...]
    pos1 = jnp.sum(oh1 * offs, axis=1, keepdims=True).astype(jnp.int32) + r1_ref[...]
    pos0_ref[...] = pos0
    pos1_ref[...] = pos1

    col = lax.broadcasted_iota(jnp.int32, (T, BM), 1) + b * BM
    p0 = (jnp.broadcast_to(pos0, (T, BM)) == col).astype(jnp.float32)
    p1 = (jnp.broadcast_to(pos1, (T, BM)) == col).astype(jnp.float32)
    tv = lax.broadcasted_iota(jnp.int32, (T, 1), 0).astype(jnp.float32)
    dn = (((0,), (0,)), ((), ()))
    tok = lax.dot_general(p0 + p1, tv, dn,
                          preferred_element_type=jnp.float32)
    tok_ref[...] = tok.astype(jnp.int32)
    wl_ref[...] = (
        lax.dot_general(p0, w0_ref[...], dn, preferred_element_type=jnp.float32)
        + lax.dot_general(p1, w1_ref[...], dn, preferred_element_type=jnp.float32))


def _meta(counts, e0, e1, r0, r1, w0, w1):
    full = pl.BlockSpec((T, 1), lambda b: (0, 0))
    return pl.pallas_call(
        _meta_body,
        grid=(NB,),
        in_specs=[pl.BlockSpec((1, 128), lambda b: (0, 0)),
                  full, full, full, full, full, full],
        out_specs=[
            pl.BlockSpec((T, 1), lambda b: (0, 0)),
            pl.BlockSpec((T, 1), lambda b: (0, 0)),
            pl.BlockSpec((128, 1), lambda b: (0, 0)),
            pl.BlockSpec((BM, 1), lambda b: (b, 0)),
            pl.BlockSpec((BM, 1), lambda b: (b, 0)),
        ],
        out_shape=[
            jax.ShapeDtypeStruct((T, 1), jnp.int32),
            jax.ShapeDtypeStruct((T, 1), jnp.int32),
            jax.ShapeDtypeStruct((128, 1), jnp.int32),
            jax.ShapeDtypeStruct((CAP, 1), jnp.int32),
            jax.ShapeDtypeStruct((CAP, 1), jnp.float32),
        ],
        compiler_params=pltpu.CompilerParams(
            dimension_semantics=("arbitrary",),
        ),
    )(counts, e0, e1, r0, r1, w0, w1)


def _sc_gather_body(tok_h, x_h, xs_h, idx0_v, idx1_v, rows0_v, rows1_v,
                    sem0, sem1):
    wid = lax.axis_index("s") * 2 + lax.axis_index("c")
    base = wid * GPW
    idx = (idx0_v, idx1_v)
    rows = (rows0_v, rows1_v)
    sems = (sem0, sem1)
    nch = GPW // GCH
    cps = [None, None]
    for ch in range(nch):
        cur = ch % 2
        b = base + ch * GCH
        pltpu.sync_copy(tok_h.at[pl.ds(b, GCH)], idx[cur])
        cps[cur] = pltpu.async_copy(x_h.at[idx[cur]], rows[cur], sems[cur])
        if ch > 0:
            prev = 1 - cur
            cps[prev].wait()
            pltpu.sync_copy(rows[prev],
                            xs_h.at[pl.ds(base + (ch - 1) * GCH, GCH)])
    last = (nch - 1) % 2
    cps[last].wait()
    pltpu.sync_copy(rows[last], xs_h.at[pl.ds(base + (nch - 1) * GCH, GCH)])


def _sc_gather(tok, x):
    mesh = plsc.VectorSubcoreMesh(core_axis_name="c", subcore_axis_name="s")
    k = functools.partial(
        pl.kernel,
        mesh=mesh,
        out_type=jax.ShapeDtypeStruct((CAP, D_MODEL), jnp.float32),
        scratch_types=[
            pltpu.VMEM((GCH,), jnp.int32),
            pltpu.VMEM((GCH,), jnp.int32),
            pltpu.VMEM((GCH, D_MODEL), jnp.float32),
            pltpu.VMEM((GCH, D_MODEL), jnp.float32),
            pltpu.SemaphoreType.DMA,
            pltpu.SemaphoreType.DMA,
        ],
        compiler_params=pltpu.CompilerParams(needs_layout_passes=False),
    )(_sc_gather_body)
    return k(tok, x)


def _ffn_body(blk_ref, xs_ref, wl_ref, wgu_ref, wdn_ref, ys_ref):
    xb = xs_ref[...].astype(jnp.bfloat16)
    gu = jnp.dot(xb, wgu_ref[0], preferred_element_type=jnp.float32)
    g = gu[:, :D_FF]
    u = gu[:, D_FF:]
    act = (g * lax.logistic(g)) * u * wl_ref[...]
    ys_ref[...] = jnp.dot(act.astype(jnp.bfloat16), wdn_ref[0],
                          preferred_element_type=jnp.float32)


def _grouped_ffn(blk_e, xs2d, wl2d, wgu, wdn):
    grid_spec = pltpu.PrefetchScalarGridSpec(
        num_scalar_prefetch=1,
        grid=(NB,),
        in_specs=[
            pl.BlockSpec((BM, D_MODEL), lambda b, be: (b, 0)),
            pl.BlockSpec((BM, 1), lambda b, be: (b, 0)),
            pl.BlockSpec((1, D_MODEL, 2 * D_FF), lambda b, be: (be[b], 0, 0)),
            pl.BlockSpec((1, D_FF, D_MODEL), lambda b, be: (be[b], 0, 0)),
        ],
        out_specs=pl.BlockSpec((BM, D_MODEL), lambda b, be: (b, 0)),
    )
    return pl.pallas_call(
        _ffn_body,
        grid_spec=grid_spec,
        out_shape=jax.ShapeDtypeStruct((CAP, D_MODEL), jnp.float32),
        compiler_params=pltpu.CompilerParams(
            dimension_semantics=("arbitrary",),
        ),
    )(blk_e, xs2d, wl2d, wgu, wdn)


def _sc_combine_body(ys_h, pos0_h, pos1_h, out_h,
                     p0_v, p1_v, rows0_v, rows1_v, sem0, sem1):
    wid = lax.axis_index("s") * 2 + lax.axis_index("c")
    for ch in range(TPW // GCH):
        b = wid * TPW + ch * GCH
        pltpu.sync_copy(pos0_h.at[pl.ds(b, GCH)], p0_v)
        pltpu.sync_copy(pos1_h.at[pl.ds(b, GCH)], p1_v)
        c0 = pltpu.async_copy(ys_h.at[p0_v], rows0_v, sem0)
        c1 = pltpu.async_copy(ys_h.at[p1_v], rows1_v, sem1)
        c0.wait()
        c1.wait()

        for i in range(GCH):
            def lane_body(j, _2):
                sl = pl.ds(j * 16, 16)
                rows0_v[i, sl] = rows0_v[i, sl] + rows1_v[i, sl]
                return 0
            lax.fori_loop(0, D_MODEL // 16, lane_body, 0)
        pltpu.sync_copy(rows0_v, out_h.at[pl.ds(b, GCH)])


def _sc_combine(ys, pos0, pos1):
    mesh = plsc.VectorSubcoreMesh(core_axis_name="c", subcore_axis_name="s")
    k = functools.partial(
        pl.kernel,
        mesh=mesh,
        out_type=jax.ShapeDtypeStruct((T, D_MODEL), jnp.float32),
        scratch_types=[
            pltpu.VMEM((GCH,), jnp.int32),
            pltpu.VMEM((GCH,), jnp.int32),
            pltpu.VMEM((GCH, D_MODEL), jnp.float32),
            pltpu.VMEM((GCH, D_MODEL), jnp.float32),
            pltpu.SemaphoreType.DMA,
            pltpu.SemaphoreType.DMA,
        ],
        compiler_params=pltpu.CompilerParams(needs_layout_passes=False),
    )(_sc_combine_body)
    return k(ys, pos0, pos1)


def kernel(hidden_states, num_global_tokens, max_num_tokens_per_gpu,
           gate_w, w_gate_up, w_down):
    e0, e1, r0, r1, w0, w1, counts = _router(hidden_states, gate_w)

    pos0, pos1, blk, tok, wl = _meta(counts, e0, e1, r0, r1, w0, w1)

    xs = _sc_gather(tok.reshape(CAP), hidden_states)

    ys = _grouped_ffn(blk.reshape(128)[:NB], xs, wl,
                      w_gate_up.astype(jnp.bfloat16),
                      w_down.astype(jnp.bfloat16))

    return _sc_combine(ys, pos0.reshape(T), pos1.reshape(T))
